# baseline (speedup 1.0000x reference)
import functools

import jax
import jax.numpy as jnp
from jax import lax
from jax.experimental import pallas as pl
from jax.experimental.pallas import tpu as pltpu
from jax.experimental.pallas import tpu_sc as plsc

_NC = 2
_NS = 16
_L = 16


def _sc_gather(flat_logits, x, B, V):
  mesh = plsc.VectorSubcoreMesh(core_axis_name="c", subcore_axis_name="s")

  @functools.partial(
      pl.kernel,
      out_type=jax.ShapeDtypeStruct((B,), jnp.float32),
      mesh=mesh,
      scratch_types=[
          pltpu.VMEM((B,), jnp.int32),
          pltpu.VMEM((B,), jnp.int32),
          pltpu.VMEM((B,), jnp.float32),
          pltpu.SemaphoreType.DMA,
      ],
  )
  def gather_k(flat_hbm, x_hbm, out_hbm, xv, idxv, pv, sem):
    wid = lax.axis_index("s") * _NC + lax.axis_index("c")

    @pl.when(wid == 0)
    def _():
      pltpu.sync_copy(x_hbm, xv)
      for i in range(B // _L):
        row = lax.iota(jnp.int32, _L) + (i * _L)
        idxv[pl.ds(i * _L, _L)] = row * V + xv[pl.ds(i * _L, _L)]
      pltpu.async_copy(flat_hbm.at[idxv], pv, sem).wait()
      pltpu.sync_copy(pv, out_hbm)

  return gather_k(flat_logits, x)


def _tc_body(V, C, logits_ref, picked_ref, out_ref, m_ref, s_ref):
  j = pl.program_id(0)
  nb = pl.num_programs(0)

  @pl.when(j == 0)
  def _():
    m_ref[...] = jnp.full(m_ref.shape, -jnp.inf, jnp.float32)
    s_ref[...] = jnp.zeros(s_ref.shape, jnp.float32)

  chunk = logits_ref[...]
  col = j * C + lax.broadcasted_iota(jnp.int32, chunk.shape, 1)
  chunk = jnp.where(col < V, chunk, -jnp.inf)
  m_old = m_ref[...]
  m_new = jnp.maximum(m_old, jnp.max(chunk, axis=1, keepdims=True))
  s_ref[...] = s_ref[...] * jnp.exp(m_old - m_new) + jnp.sum(
      jnp.exp(chunk - m_new), axis=1, keepdims=True)
  m_ref[...] = m_new

  @pl.when(j == nb - 1)
  def _():
    lse = m_ref[...] + jnp.log(s_ref[...])
    out_ref[...] = (jnp.sum(picked_ref[...]) - jnp.sum(lse)).reshape(1, 1)


def kernel(logits, x):
  B, V = logits.shape
  x = x.astype(jnp.int32)

  picked = _sc_gather(logits.reshape(-1), x, B, V)

  C = 2048
  K = pl.cdiv(V, C)
  out = pl.pallas_call(
      functools.partial(_tc_body, V, C),
      grid=(K,),
      in_specs=[
          pl.BlockSpec((B, C), lambda j: (0, j)),
          pl.BlockSpec((1, B), lambda j: (0, 0)),
      ],
      out_specs=pl.BlockSpec((1, 1), lambda j: (0, 0)),
      out_shape=jax.ShapeDtypeStruct((1, 1), jnp.float32),
      scratch_shapes=[
          pltpu.VMEM((B, 1), jnp.float32),
          pltpu.VMEM((B, 1), jnp.float32),
      ],
  )(logits, picked.reshape(1, B))
  return out[0, 0]


# speedup vs baseline: 1.0639x; 1.0639x over previous
import functools

import jax
import jax.numpy as jnp
from jax import lax
from jax.experimental import pallas as pl
from jax.experimental.pallas import tpu as pltpu
from jax.experimental.pallas import tpu_sc as plsc

_NC = 2
_NS = 16
_L = 16


def _sc_gather(flat_logits, x, B, V):
  mesh = plsc.VectorSubcoreMesh(core_axis_name="c", subcore_axis_name="s")

  @functools.partial(
      pl.kernel,
      out_type=jax.ShapeDtypeStruct((B,), jnp.float32),
      mesh=mesh,
      scratch_types=[
          pltpu.VMEM((B,), jnp.int32),
          pltpu.VMEM((B,), jnp.int32),
          pltpu.VMEM((B,), jnp.float32),
          pltpu.SemaphoreType.DMA,
      ],
  )
  def gather_k(flat_hbm, x_hbm, out_hbm, xv, idxv, pv, sem):
    wid = lax.axis_index("s") * _NC + lax.axis_index("c")

    @pl.when(wid == 0)
    def _():
      pltpu.sync_copy(x_hbm, xv)
      for i in range(B // _L):
        row = lax.iota(jnp.int32, _L) + (i * _L)
        idxv[pl.ds(i * _L, _L)] = row * V + xv[pl.ds(i * _L, _L)]
      pltpu.async_copy(flat_hbm.at[idxv], pv, sem).wait()
      pltpu.sync_copy(pv, out_hbm)

  return gather_k(flat_logits, x)


def _tc_body(logits_ref, picked_ref, out_ref):
  j = pl.program_id(0)
  chunk = logits_ref[...]
  m = jnp.max(chunk, axis=1, keepdims=True)
  s = jnp.sum(jnp.exp(chunk - m), axis=1, keepdims=True)
  part = -jnp.sum(m + jnp.log(s))

  @pl.when(j == 0)
  def _():
    out_ref[...] = (jnp.sum(picked_ref[...]) + part).reshape(1, 1)

  @pl.when(j > 0)
  def _():
    out_ref[...] += part.reshape(1, 1)


def kernel(logits, x):
  B, V = logits.shape
  x = x.astype(jnp.int32)

  picked = _sc_gather(logits.reshape(-1), x, B, V)

  R = 8
  out = pl.pallas_call(
      _tc_body,
      grid=(B // R,),
      in_specs=[
          pl.BlockSpec((R, V), lambda j: (j, 0)),
          pl.BlockSpec((1, B), lambda j: (0, 0)),
      ],
      out_specs=pl.BlockSpec((1, 1), lambda j: (0, 0)),
      out_shape=jax.ShapeDtypeStruct((1, 1), jnp.float32),
  )(logits, picked.reshape(1, B))
  return out[0, 0]
